# baseline (speedup 1.0000x reference)
import functools
import jax
import jax.numpy as jnp
from jax import lax
from jax.experimental import pallas as pl
from jax.experimental.pallas import tpu as pltpu
from jax.experimental.pallas import tpu_sc as plsc

ROWS_PER_CHUNK = 16
D_LANES = 16


def _sc_add(B, S, D):
    NW = 32
    rows_total = B * S
    rows_per_w = rows_total // NW
    n_chunks = rows_per_w // ROWS_PER_CHUNK
    mesh = plsc.VectorSubcoreMesh(core_axis_name="c", subcore_axis_name="s")

    @functools.partial(
        pl.kernel,
        out_type=jax.ShapeDtypeStruct((B, S, D), jnp.float32),
        mesh=mesh,
        scratch_types=[
            pltpu.VMEM((ROWS_PER_CHUNK, D), jnp.float32),
            pltpu.VMEM((ROWS_PER_CHUNK, D), jnp.float32),
        ],
    )
    def k(x_hbm, pos_hbm, out_hbm, xbuf, posbuf):
        wid = lax.axis_index("s") * 2 + lax.axis_index("c")
        row0 = wid * rows_per_w
        b = row0 // S
        s_base = row0 % S

        def chunk(c, _):
            s0 = s_base + c * ROWS_PER_CHUNK
            pltpu.sync_copy(x_hbm.at[b, pl.ds(s0, ROWS_PER_CHUNK), :], xbuf)
            pltpu.sync_copy(pos_hbm.at[pl.ds(s0, ROWS_PER_CHUNK), :], posbuf)
            for r in range(ROWS_PER_CHUNK):
                def lane(j, _2):
                    sl = pl.ds(j * D_LANES, D_LANES)
                    xbuf[r, sl] = xbuf[r, sl] + posbuf[r, sl]
                    return 0
                lax.fori_loop(0, D // D_LANES, lane, 0)
            pltpu.sync_copy(xbuf, out_hbm.at[b, pl.ds(s0, ROWS_PER_CHUNK), :])
            return 0

        lax.fori_loop(0, n_chunks, chunk, 0)

    return k


def kernel(x, pos_embedding):
    B, S, D = x.shape
    pos = pos_embedding[:S]
    return _sc_add(B, S, D)(x, pos)


# speedup vs baseline: 3.3504x; 3.3504x over previous
import functools
import jax
import jax.numpy as jnp
from jax import lax
from jax.experimental import pallas as pl
from jax.experimental.pallas import tpu as pltpu
from jax.experimental.pallas import tpu_sc as plsc

ROWS = 16
NBUF = 3
L = 16


def _sc_add(B, S, D):
    NW = 32
    rows_per_w = (B * S) // NW
    n_chunks = rows_per_w // ROWS
    mesh = plsc.VectorSubcoreMesh(core_axis_name="c", subcore_axis_name="s")

    @functools.partial(
        pl.kernel,
        out_type=jax.ShapeDtypeStruct((B, S, D), jnp.float32),
        mesh=mesh,
        scratch_types=[
            pltpu.VMEM((NBUF, ROWS, D), jnp.float32),
            pltpu.VMEM((NBUF, ROWS, D), jnp.float32),
            pltpu.SemaphoreType.DMA((NBUF,)),
            pltpu.SemaphoreType.DMA((NBUF,)),
            pltpu.SemaphoreType.DMA((NBUF,)),
        ],
    )
    def k(x_hbm, pos_hbm, out_hbm, xbuf, posbuf, xsem, psem, osem):
        wid = lax.axis_index("s") * 2 + lax.axis_index("c")
        row0 = wid * rows_per_w
        b = row0 // S
        s_base = row0 % S

        def start_loads(c):
            slot = c % NBUF
            s0 = s_base + c * ROWS
            pltpu.async_copy(
                x_hbm.at[b, pl.ds(s0, ROWS), :], xbuf.at[slot], xsem.at[slot])
            pltpu.async_copy(
                pos_hbm.at[pl.ds(s0, ROWS), :], posbuf.at[slot], psem.at[slot])

        def wait_out(c):
            slot = c % NBUF
            s0 = s_base + c * ROWS
            pltpu.make_async_copy(
                xbuf.at[slot], out_hbm.at[b, pl.ds(s0, ROWS), :],
                osem.at[slot]).wait()

        start_loads(0)

        def chunk(c, _):
            slot = c % NBUF
            s0 = s_base + c * ROWS

            @pl.when(c + 1 < n_chunks)
            def _prefetch():
                @pl.when(c >= NBUF - 1)
                def _drain():
                    wait_out(c - (NBUF - 1))
                start_loads(c + 1)

            pltpu.make_async_copy(
                x_hbm.at[b, pl.ds(s0, ROWS), :], xbuf.at[slot],
                xsem.at[slot]).wait()
            pltpu.make_async_copy(
                pos_hbm.at[pl.ds(s0, ROWS), :], posbuf.at[slot],
                psem.at[slot]).wait()

            @plsc.parallel_loop(0, (ROWS * D) // L, 1, unroll=8)
            def _add(j):
                r = j >> 6
                sl = pl.ds((j & 63) * L, L)
                xbuf[slot, r, sl] = xbuf[slot, r, sl] + posbuf[slot, r, sl]

            pltpu.async_copy(
                xbuf.at[slot], out_hbm.at[b, pl.ds(s0, ROWS), :],
                osem.at[slot])
            return 0

        lax.fori_loop(0, n_chunks, chunk, 0)

        for c in range(n_chunks - NBUF, n_chunks):
            wait_out(c)

    return k


def kernel(x, pos_embedding):
    B, S, D = x.shape
    pos = pos_embedding[:S]
    return _sc_add(B, S, D)(x, pos)


# speedup vs baseline: 5.8029x; 1.7320x over previous
import jax
import jax.numpy as jnp
from jax.experimental import pallas as pl
from jax.experimental.pallas import tpu as pltpu

S_BLK = 2048


def _add_kernel(x_ref, pos_ref, out_ref):
    out_ref[0, :, :] = x_ref[0, :, :] + pos_ref[...]


def kernel(x, pos_embedding):
    B, S, D = x.shape
    pos = pos_embedding[:S]
    grid = (S // S_BLK, B)
    return pl.pallas_call(
        _add_kernel,
        grid=grid,
        in_specs=[
            pl.BlockSpec((1, S_BLK, D), lambda i, b: (b, i, 0)),
            pl.BlockSpec((S_BLK, D), lambda i, b: (i, 0)),
        ],
        out_specs=pl.BlockSpec((1, S_BLK, D), lambda i, b: (b, i, 0)),
        out_shape=jax.ShapeDtypeStruct((B, S, D), x.dtype),
        compiler_params=pltpu.CompilerParams(
            dimension_semantics=("parallel", "parallel"),
            vmem_limit_bytes=64 * 1024 * 1024,
        ),
    )(x, pos)
